# baseline (speedup 1.0000x reference)
import jax
import jax.numpy as jnp
import numpy as np
from jax import lax
from jax.experimental import pallas as pl
from jax.experimental.pallas import tpu as pltpu
from jax.experimental.pallas import tpu_sc as plsc

NC = 2
NS = 16
L = 16
NW = NC * NS

VOCAB = 1000000
EMBD = 64
B = 16384

CW = 8192
NBH = 61
HM = NBH * CW
RES = 2 * HM
TROWS = HM // 2 + CW // 2
PW = 128

BPW = B // NW
GCH = 128
NG = BPW // GCH


def _tc_pack(xa, xb, o, z):
    ta = jnp.transpose(xa[...]).astype(jnp.bfloat16)
    tb = jnp.transpose(xb[...]).astype(jnp.bfloat16)
    z[...] = jnp.concatenate([ta, tb], axis=1)
    o[...] = z.bitcast(jnp.uint32)[...]


def _pack_table(emb_t):
    specs = [
        pl.BlockSpec(
            (EMBD, CW),
            lambda c, h=h: (0, jnp.where(c < NBH, h * NBH + c, 2 * NBH)))
        for h in range(2)
    ]
    return pl.pallas_call(
        _tc_pack,
        grid=(NBH + 1,),
        in_specs=specs,
        out_specs=pl.BlockSpec((CW // 2, PW), lambda c: (c, 0)),
        out_shape=jax.ShapeDtypeStruct((TROWS, PW), jnp.uint32),
        scratch_shapes=[pltpu.VMEM((CW, 2 * EMBD), jnp.bfloat16)],
    )(emb_t, emb_t)


def _sc_kernel(focus_hbm, ctx_hbm, table_hbm, out_hbm,
               fidx_v, cidx_v, fr_v, fp_v, cr_v, cp_v,
               frows_v, crows_v, out_v, sem):
    wid = lax.axis_index("s") * NC + lax.axis_index("c")
    base = wid * BPW

    pltpu.sync_copy(focus_hbm.at[pl.ds(wid * NG, NG)], fidx_v)
    pltpu.sync_copy(ctx_hbm.at[pl.ds(wid * NG, NG)], cidx_v)

    def decompose(idx_v, r_v, p_v):
        for k in range(NG):
            for m in range(GCH // L):
                v = idx_v[k, pl.ds(m * L, L)]
                h3 = v >= RES
                h2 = (v >= HM).astype(jnp.int32)
                rm = (v - h2 * HM) >> 1
                rr = ((v - RES) >> 1) + HM // 2
                sh = 16 - (v & 1) * 16
                r_v[k, pl.ds(m * L, L)] = jnp.where(h3, rr, rm)
                p_v[k, pl.ds(m * L, L)] = jnp.where(h3, sh, h2 * 64 + sh)

    decompose(fidx_v, fr_v, fp_v)
    decompose(cidx_v, cr_v, cp_v)

    lane = lax.iota(jnp.int32, L)

    def expand(rows_v, i, cb, sh):
        vals = []
        for k in range(EMBD // L):
            u = rows_v[i, pl.ds(cb + k * L, L)]
            vals.append(plsc.bitcast((u << sh) & np.uint32(0xFFFF0000),
                                     jnp.float32))
        return vals

    for ch in range(NG):
        gf = pltpu.async_copy(table_hbm.at[fr_v.at[ch]], frows_v, sem)
        gc = pltpu.async_copy(table_hbm.at[cr_v.at[ch]], crows_v, sem)
        gf.wait()
        gc.wait()

        def block(bb, carry, ch=ch):
            row0 = bb * L
            res = jnp.zeros((L,), jnp.float32)
            fpars = fp_v[ch, pl.ds(row0, L)]
            cpars = cp_v[ch, pl.ds(row0, L)]
            for u in range(L):
                i = row0 + u
                fpar = fpars[u]
                cpar = cpars[u]
                fv = expand(frows_v, i, fpar & 64, (fpar & 16).astype(jnp.uint32))
                cv = expand(crows_v, i, cpar & 64, (cpar & 16).astype(jnp.uint32))
                acc = fv[0] * cv[0]
                for k in range(1, EMBD // L):
                    acc = acc + fv[k] * cv[k]
                res = jnp.where(lane == u, jnp.sum(acc), res)
            out_v[pl.ds(ch * GCH + row0, L)] = res
            return carry

        lax.fori_loop(0, GCH // L, block, 0)

    pltpu.sync_copy(out_v, out_hbm.at[pl.ds(base, BPW)])


@jax.jit
def kernel(focus, context, embeddings):
    mesh = plsc.VectorSubcoreMesh(
        core_axis_name="c", subcore_axis_name="s",
        num_cores=NC, num_subcores=NS)
    k = pl.kernel(
        _sc_kernel,
        out_type=jax.ShapeDtypeStruct((B,), jnp.float32),
        mesh=mesh,
        compiler_params=pltpu.CompilerParams(needs_layout_passes=False,
                                             use_tc_tiling_on_sc=False),
        scratch_types=[
            pltpu.VMEM((NG, GCH), jnp.int32),
            pltpu.VMEM((NG, GCH), jnp.int32),
            pltpu.VMEM((NG, GCH), jnp.int32),
            pltpu.VMEM((NG, GCH), jnp.int32),
            pltpu.VMEM((NG, GCH), jnp.int32),
            pltpu.VMEM((NG, GCH), jnp.int32),
            pltpu.VMEM((GCH, PW), jnp.uint32),
            pltpu.VMEM((GCH, PW), jnp.uint32),
            pltpu.VMEM((BPW,), jnp.float32),
            pltpu.SemaphoreType.DMA,
        ],
    )
    table = _pack_table(embeddings.T)
    focus2d = focus.reshape(B // GCH, GCH)
    ctx2d = context.reshape(B // GCH, GCH)
    return k(focus2d, ctx2d, table)


# speedup vs baseline: 1.0741x; 1.0741x over previous
import jax
import jax.numpy as jnp
import numpy as np
from jax import lax
from jax.experimental import pallas as pl
from jax.experimental.pallas import tpu as pltpu
from jax.experimental.pallas import tpu_sc as plsc

NC = 2
NS = 16
L = 16
NW = NC * NS

VOCAB = 1000000
EMBD = 64
B = 16384

CW = 16384
NBH = 30
HM = NBH * CW
RES = 2 * HM
NR = -(-(VOCAB - RES) // CW)
TROWS = HM // 2 + NR * CW // 2
PW = 128

BPW = B // NW
GCH = 128
NG = BPW // GCH


def _tc_pack(xa, xb, o, z):
    ta = jnp.transpose(xa[...]).astype(jnp.bfloat16)
    tb = jnp.transpose(xb[...]).astype(jnp.bfloat16)
    z[...] = jnp.concatenate([ta, tb], axis=1)
    o[...] = z.bitcast(jnp.uint32)[...]


def _pack_table(emb_t):
    specs = [
        pl.BlockSpec(
            (EMBD, CW),
            lambda c, h=h: (0, jnp.where(c < NBH, h * NBH + c, NBH + c)))
        for h in range(2)
    ]
    return pl.pallas_call(
        _tc_pack,
        grid=(NBH + NR,),
        in_specs=specs,
        out_specs=pl.BlockSpec((CW // 2, PW), lambda c: (c, 0)),
        out_shape=jax.ShapeDtypeStruct((TROWS, PW), jnp.uint32),
        scratch_shapes=[pltpu.VMEM((CW, 2 * EMBD), jnp.bfloat16)],
    )(emb_t, emb_t)


def _sc_kernel(focus_hbm, ctx_hbm, table_hbm, out_hbm,
               fidx_v, cidx_v, fr_v, fp_v, cr_v, cp_v,
               frows_v, crows_v, out_v, sem):
    wid = lax.axis_index("s") * NC + lax.axis_index("c")
    base = wid * BPW

    pltpu.sync_copy(focus_hbm.at[pl.ds(wid * NG, NG)], fidx_v)
    pltpu.sync_copy(ctx_hbm.at[pl.ds(wid * NG, NG)], cidx_v)

    def decompose(idx_v, r_v, p_v):
        for k in range(NG):
            for m in range(GCH // L):
                v = idx_v[k, pl.ds(m * L, L)]
                h3 = v >= RES
                h2 = (v >= HM).astype(jnp.int32)
                rm = (v - h2 * HM) >> 1
                rr = ((v - RES) >> 1) + HM // 2
                sh = 16 - (v & 1) * 16
                r_v[k, pl.ds(m * L, L)] = jnp.where(h3, rr, rm)
                p_v[k, pl.ds(m * L, L)] = jnp.where(h3, sh, h2 * 64 + sh)

    decompose(fidx_v, fr_v, fp_v)
    decompose(cidx_v, cr_v, cp_v)

    lane = lax.iota(jnp.int32, L)

    def expand(rows_v, i, cb, sh):
        vals = []
        for k in range(EMBD // L):
            u = rows_v[i, pl.ds(cb + k * L, L)]
            vals.append(plsc.bitcast((u << sh) & np.uint32(0xFFFF0000),
                                     jnp.float32))
        return vals

    for ch in range(NG):
        gf = pltpu.async_copy(table_hbm.at[fr_v.at[ch]], frows_v, sem)
        gc = pltpu.async_copy(table_hbm.at[cr_v.at[ch]], crows_v, sem)
        gf.wait()
        gc.wait()

        def block(bb, carry, ch=ch):
            row0 = bb * L
            res = jnp.zeros((L,), jnp.float32)
            fpars = fp_v[ch, pl.ds(row0, L)]
            cpars = cp_v[ch, pl.ds(row0, L)]
            for u in range(L):
                i = row0 + u
                fpar = fpars[u]
                cpar = cpars[u]
                fv = expand(frows_v, i, fpar & 64, (fpar & 16).astype(jnp.uint32))
                cv = expand(crows_v, i, cpar & 64, (cpar & 16).astype(jnp.uint32))
                acc = fv[0] * cv[0]
                for k in range(1, EMBD // L):
                    acc = acc + fv[k] * cv[k]
                res = jnp.where(lane == u, jnp.sum(acc), res)
            out_v[pl.ds(ch * GCH + row0, L)] = res
            return carry

        lax.fori_loop(0, GCH // L, block, 0)

    pltpu.sync_copy(out_v, out_hbm.at[pl.ds(base, BPW)])


@jax.jit
def kernel(focus, context, embeddings):
    mesh = plsc.VectorSubcoreMesh(
        core_axis_name="c", subcore_axis_name="s",
        num_cores=NC, num_subcores=NS)
    k = pl.kernel(
        _sc_kernel,
        out_type=jax.ShapeDtypeStruct((B,), jnp.float32),
        mesh=mesh,
        compiler_params=pltpu.CompilerParams(needs_layout_passes=False,
                                             use_tc_tiling_on_sc=False),
        scratch_types=[
            pltpu.VMEM((NG, GCH), jnp.int32),
            pltpu.VMEM((NG, GCH), jnp.int32),
            pltpu.VMEM((NG, GCH), jnp.int32),
            pltpu.VMEM((NG, GCH), jnp.int32),
            pltpu.VMEM((NG, GCH), jnp.int32),
            pltpu.VMEM((NG, GCH), jnp.int32),
            pltpu.VMEM((GCH, PW), jnp.uint32),
            pltpu.VMEM((GCH, PW), jnp.uint32),
            pltpu.VMEM((BPW,), jnp.float32),
            pltpu.SemaphoreType.DMA,
        ],
    )
    table = _pack_table(embeddings.T)
    focus2d = focus.reshape(B // GCH, GCH)
    ctx2d = context.reshape(B // GCH, GCH)
    return k(focus2d, ctx2d, table)


# speedup vs baseline: 1.0903x; 1.0151x over previous
import jax
import jax.numpy as jnp
import numpy as np
from jax import lax
from jax.experimental import pallas as pl
from jax.experimental.pallas import tpu as pltpu
from jax.experimental.pallas import tpu_sc as plsc

NC = 2
NS = 16
L = 16
NW = NC * NS

VOCAB = 1000000
EMBD = 64
B = 16384

CW = 16384
NBH = 30
HM = NBH * CW
RES = 2 * HM
NR = -(-(VOCAB - RES) // CW)
TROWS = HM // 2 + NR * CW // 2
PW = 128

BPW = B // NW
GCH = 128
NG = BPW // GCH


def _tc_pack(xa, xb, o, z):
    ta = jnp.transpose(xa[...]).astype(jnp.bfloat16)
    tb = jnp.transpose(xb[...]).astype(jnp.bfloat16)
    z[...] = jnp.concatenate([ta, tb], axis=1)
    o[...] = z.bitcast(jnp.uint32)[...]


def _pack_table(emb_t):
    specs = [
        pl.BlockSpec(
            (EMBD, CW),
            lambda c, h=h: (0, jnp.where(c < NBH, h * NBH + c, NBH + c)))
        for h in range(2)
    ]
    return pl.pallas_call(
        _tc_pack,
        grid=(NBH + NR,),
        in_specs=specs,
        out_specs=pl.BlockSpec((CW // 2, PW), lambda c: (c, 0)),
        out_shape=jax.ShapeDtypeStruct((TROWS, PW), jnp.uint32),
        scratch_shapes=[pltpu.VMEM((CW, 2 * EMBD), jnp.bfloat16)],
    )(emb_t, emb_t)


def _sc_kernel(focus_hbm, ctx_hbm, table_hbm, out_hbm,
               fidx_v, cidx_v, fr_v, fp_v, cr_v, cp_v,
               frows_v, crows_v, out_v, sem0, sem1):
    wid = lax.axis_index("s") * NC + lax.axis_index("c")
    base = wid * BPW

    pltpu.sync_copy(focus_hbm.at[pl.ds(wid * NG, NG)], fidx_v)
    pltpu.sync_copy(ctx_hbm.at[pl.ds(wid * NG, NG)], cidx_v)

    def decompose(idx_v, r_v, p_v):
        for k in range(NG):
            for m in range(GCH // L):
                v = idx_v[k, pl.ds(m * L, L)]
                h3 = v >= RES
                h2 = (v >= HM).astype(jnp.int32)
                rm = (v - h2 * HM) >> 1
                rr = ((v - RES) >> 1) + HM // 2
                sh = 16 - (v & 1) * 16
                r_v[k, pl.ds(m * L, L)] = jnp.where(h3, rr, rm)
                p_v[k, pl.ds(m * L, L)] = jnp.where(h3, sh, h2 * 64 + sh)

    decompose(fidx_v, fr_v, fp_v)
    decompose(cidx_v, cr_v, cp_v)

    lane = lax.iota(jnp.int32, L)

    def expand(rows_v, i, cb, sh):
        vals = []
        for k in range(EMBD // L):
            u = rows_v[i, pl.ds(cb + k * L, L)]
            vals.append(plsc.bitcast((u << sh) & np.uint32(0xFFFF0000),
                                     jnp.float32))
        return vals

    def fire(ch):
        sl = pl.ds((ch % 2) * GCH, GCH)
        sem = sem1 if ch % 2 else sem0
        return (pltpu.async_copy(table_hbm.at[fr_v.at[ch]], frows_v.at[sl], sem),
                pltpu.async_copy(table_hbm.at[cr_v.at[ch]], crows_v.at[sl], sem))

    pending = fire(0)
    for ch in range(NG):
        nxt = fire(ch + 1) if ch + 1 < NG else None
        for c in pending:
            c.wait()
        pending = nxt
        buf0 = (ch % 2) * GCH

        def block(bb, carry, ch=ch, buf0=buf0):
            row0 = bb * L
            res = jnp.zeros((L,), jnp.float32)
            fpars = fp_v[ch, pl.ds(row0, L)]
            cpars = cp_v[ch, pl.ds(row0, L)]
            for u in range(L):
                i = buf0 + row0 + u
                fpar = fpars[u]
                cpar = cpars[u]
                fv = expand(frows_v, i, fpar & 64, (fpar & 16).astype(jnp.uint32))
                cv = expand(crows_v, i, cpar & 64, (cpar & 16).astype(jnp.uint32))
                acc = fv[0] * cv[0]
                for k in range(1, EMBD // L):
                    acc = acc + fv[k] * cv[k]
                res = jnp.where(lane == u, jnp.sum(acc), res)
            out_v[pl.ds(ch * GCH + row0, L)] = res
            return carry

        lax.fori_loop(0, GCH // L, block, 0)

    pltpu.sync_copy(out_v, out_hbm.at[pl.ds(base, BPW)])


@jax.jit
def kernel(focus, context, embeddings):
    mesh = plsc.VectorSubcoreMesh(
        core_axis_name="c", subcore_axis_name="s",
        num_cores=NC, num_subcores=NS)
    k = pl.kernel(
        _sc_kernel,
        out_type=jax.ShapeDtypeStruct((B,), jnp.float32),
        mesh=mesh,
        compiler_params=pltpu.CompilerParams(needs_layout_passes=False,
                                             use_tc_tiling_on_sc=False),
        scratch_types=[
            pltpu.VMEM((NG, GCH), jnp.int32),
            pltpu.VMEM((NG, GCH), jnp.int32),
            pltpu.VMEM((NG, GCH), jnp.int32),
            pltpu.VMEM((NG, GCH), jnp.int32),
            pltpu.VMEM((NG, GCH), jnp.int32),
            pltpu.VMEM((NG, GCH), jnp.int32),
            pltpu.VMEM((2 * GCH, PW), jnp.uint32),
            pltpu.VMEM((2 * GCH, PW), jnp.uint32),
            pltpu.VMEM((BPW,), jnp.float32),
            pltpu.SemaphoreType.DMA,
            pltpu.SemaphoreType.DMA,
        ],
    )
    table = _pack_table(embeddings.T)
    focus2d = focus.reshape(B // GCH, GCH)
    ctx2d = context.reshape(B // GCH, GCH)
    return k(focus2d, ctx2d, table)


# speedup vs baseline: 1.1386x; 1.0443x over previous
import jax
import jax.numpy as jnp
import numpy as np
from jax import lax
from jax.experimental import pallas as pl
from jax.experimental.pallas import tpu as pltpu
from jax.experimental.pallas import tpu_sc as plsc

NC = 2
NS = 16
L = 16
NW = NC * NS

VOCAB = 1000000
EMBD = 64
B = 16384

CW = 24576
NBH = 20
HM = NBH * CW
RES = 2 * HM
NR = -(-(VOCAB - RES) // CW)
TROWS = HM // 2 + NR * CW // 2
PW = 128

BPW = B // NW
GCH = 128
NG = BPW // GCH


def _tc_pack(xa, xb, o, z):
    ta = jnp.transpose(xa[...]).astype(jnp.bfloat16)
    tb = jnp.transpose(xb[...]).astype(jnp.bfloat16)
    z[...] = jnp.concatenate([ta, tb], axis=1)
    o[...] = z.bitcast(jnp.uint32)[...]


def _pack_table(emb_t):
    specs = [
        pl.BlockSpec(
            (EMBD, CW),
            lambda c, h=h: (0, jnp.where(c < NBH, h * NBH + c, NBH + c)))
        for h in range(2)
    ]
    return pl.pallas_call(
        _tc_pack,
        grid=(NBH + NR,),
        in_specs=specs,
        out_specs=pl.BlockSpec((CW // 2, PW), lambda c: (c, 0)),
        out_shape=jax.ShapeDtypeStruct((TROWS, PW), jnp.uint32),
        scratch_shapes=[pltpu.VMEM((CW, 2 * EMBD), jnp.bfloat16)],
    )(emb_t, emb_t)


def _sc_kernel(focus_hbm, ctx_hbm, table_hbm, out_hbm,
               fidx_v, cidx_v, fr_v, fp_v, cr_v, cp_v,
               frows_v, crows_v, out_v, sem0, sem1):
    wid = lax.axis_index("s") * NC + lax.axis_index("c")
    base = wid * BPW

    pltpu.sync_copy(focus_hbm.at[pl.ds(wid * NG, NG)], fidx_v)
    pltpu.sync_copy(ctx_hbm.at[pl.ds(wid * NG, NG)], cidx_v)

    def decompose(idx_v, r_v, p_v):
        for k in range(NG):
            for m in range(GCH // L):
                v = idx_v[k, pl.ds(m * L, L)]
                h3 = v >= RES
                h2 = (v >= HM).astype(jnp.int32)
                rm = (v - h2 * HM) >> 1
                rr = ((v - RES) >> 1) + HM // 2
                sh = 16 - (v & 1) * 16
                r_v[k, pl.ds(m * L, L)] = jnp.where(h3, rr, rm)
                p_v[k, pl.ds(m * L, L)] = jnp.where(h3, sh, h2 * 64 + sh)

    decompose(fidx_v, fr_v, fp_v)
    decompose(cidx_v, cr_v, cp_v)

    lane = lax.iota(jnp.int32, L)

    def expand(rows_v, i, cb, sh):
        vals = []
        for k in range(EMBD // L):
            u = rows_v[i, pl.ds(cb + k * L, L)]
            vals.append(plsc.bitcast((u << sh) & np.uint32(0xFFFF0000),
                                     jnp.float32))
        return vals

    def fire(ch):
        sl = pl.ds((ch % 2) * GCH, GCH)
        sem = sem1 if ch % 2 else sem0
        return (pltpu.async_copy(table_hbm.at[fr_v.at[ch]], frows_v.at[sl], sem),
                pltpu.async_copy(table_hbm.at[cr_v.at[ch]], crows_v.at[sl], sem))

    pending = fire(0)
    for ch in range(NG):
        nxt = fire(ch + 1) if ch + 1 < NG else None
        for c in pending:
            c.wait()
        pending = nxt
        buf0 = (ch % 2) * GCH

        def block(bb, carry, ch=ch, buf0=buf0):
            row0 = bb * L
            res = jnp.zeros((L,), jnp.float32)
            fpars = fp_v[ch, pl.ds(row0, L)]
            cpars = cp_v[ch, pl.ds(row0, L)]
            for u in range(L):
                i = buf0 + row0 + u
                fpar = fpars[u]
                cpar = cpars[u]
                fv = expand(frows_v, i, fpar & 64, (fpar & 16).astype(jnp.uint32))
                cv = expand(crows_v, i, cpar & 64, (cpar & 16).astype(jnp.uint32))
                acc = fv[0] * cv[0]
                for k in range(1, EMBD // L):
                    acc = acc + fv[k] * cv[k]
                res = jnp.where(lane == u, jnp.sum(acc), res)
            out_v[pl.ds(ch * GCH + row0, L)] = res
            return carry

        lax.fori_loop(0, GCH // L, block, 0)

    pltpu.sync_copy(out_v, out_hbm.at[pl.ds(base, BPW)])


@jax.jit
def kernel(focus, context, embeddings):
    mesh = plsc.VectorSubcoreMesh(
        core_axis_name="c", subcore_axis_name="s",
        num_cores=NC, num_subcores=NS)
    k = pl.kernel(
        _sc_kernel,
        out_type=jax.ShapeDtypeStruct((B,), jnp.float32),
        mesh=mesh,
        compiler_params=pltpu.CompilerParams(needs_layout_passes=False,
                                             use_tc_tiling_on_sc=False),
        scratch_types=[
            pltpu.VMEM((NG, GCH), jnp.int32),
            pltpu.VMEM((NG, GCH), jnp.int32),
            pltpu.VMEM((NG, GCH), jnp.int32),
            pltpu.VMEM((NG, GCH), jnp.int32),
            pltpu.VMEM((NG, GCH), jnp.int32),
            pltpu.VMEM((NG, GCH), jnp.int32),
            pltpu.VMEM((2 * GCH, PW), jnp.uint32),
            pltpu.VMEM((2 * GCH, PW), jnp.uint32),
            pltpu.VMEM((BPW,), jnp.float32),
            pltpu.SemaphoreType.DMA,
            pltpu.SemaphoreType.DMA,
        ],
    )
    table = _pack_table(embeddings.T)
    focus2d = focus.reshape(B // GCH, GCH)
    ctx2d = context.reshape(B // GCH, GCH)
    return k(focus2d, ctx2d, table)
